# baseline (speedup 1.0000x reference)
import functools

import jax
import jax.numpy as jnp
from jax import lax
from jax.experimental import pallas as pl
from jax.experimental.pallas import tpu as pltpu
from jax.experimental.pallas import tpu_sc as plsc

BATCH = 16384
NUM_FIELDS = 26
FIELD_SIZE = 100000

NC = 2
NS = 16
NW = NC * NS
B_PER_W = BATCH // NW
N_IDX = B_PER_W * NUM_FIELDS
GCHUNK = 128
N_CHUNK = N_IDX // GCHUNK
CPF = B_PER_W // GCHUNK


def _body(x_ref, table_ref, out_ref, x_v, idx_v, rub0, rub1, out_v, s0, s1):
    wid = lax.axis_index("s") * NC + lax.axis_index("c")
    base = wid * N_IDX

    pltpu.sync_copy(x_ref.at[pl.ds(base, N_IDX)], x_v)

    lanes = lax.iota(jnp.int32, 16)
    lanes26 = lanes * NUM_FIELDS
    zeros16 = jnp.zeros((16,), jnp.int32)

    def build(t, _):
        f = t // (B_PER_W // 16)
        c2 = t % (B_PER_W // 16)
        xpos = lanes26 + (c2 * 16 * NUM_FIELDS + f)
        xv = plsc.load_gather(x_v, [xpos])
        idx_v[pl.ds(t * 16, 16)] = xv + f * FIELD_SIZE
        return 0

    lax.fori_loop(0, NUM_FIELDS * (B_PER_W // 16), build, 0, unroll=4)

    def zero(c2, _):
        out_v[pl.ds(c2 * 16, 16)] = jnp.zeros((16,), jnp.float32)
        return 0

    lax.fori_loop(0, B_PER_W // 16, zero, 0)

    tab1d = table_ref.at[0]

    def start(k, rub, sem):
        pltpu.async_copy(
            tab1d.at[idx_v.at[pl.ds(k * GCHUNK, GCHUNK)]], rub, sem
        )

    def wait(k, rub, sem):
        pltpu.make_async_copy(
            tab1d.at[idx_v.at[pl.ds(k * GCHUNK, GCHUNK)]], rub, sem
        ).wait()

    def process(k, rub):
        j0 = (k % CPF) * GCHUNK
        for i in range(GCHUNK // 16):
            v = rub[pl.ds(i * 16, 16)]
            o = pl.ds(j0 + i * 16, 16)
            out_v[o] = out_v[o] + v

    start(0, rub0, s0)

    def gloop(k2, _):
        k = k2 * 2
        start(k + 1, rub1, s1)
        wait(k, rub0, s0)
        process(k, rub0)

        @pl.when(k2 < N_CHUNK // 2 - 1)
        def _():
            start(k + 2, rub0, s0)

        wait(k + 1, rub1, s1)
        process(k + 1, rub1)
        return 0

    lax.fori_loop(0, N_CHUNK // 2, gloop, 0)

    pltpu.sync_copy(out_v, out_ref.at[pl.ds(wid * B_PER_W, B_PER_W)])


@jax.jit
def kernel(x, table, bias):
    mesh = plsc.VectorSubcoreMesh(core_axis_name="c", subcore_axis_name="s")
    k = pl.kernel(
        _body,
        out_type=jax.ShapeDtypeStruct((BATCH,), jnp.float32),
        mesh=mesh,
        compiler_params=pltpu.CompilerParams(
            needs_layout_passes=False, use_tc_tiling_on_sc=False
        ),
        scratch_types=[
            pltpu.VMEM((N_IDX,), jnp.int32),
            pltpu.VMEM((N_IDX,), jnp.int32),
            pltpu.VMEM((GCHUNK,), jnp.float32),
            pltpu.VMEM((GCHUNK,), jnp.float32),
            pltpu.VMEM((B_PER_W,), jnp.float32),
            pltpu.SemaphoreType.DMA,
            pltpu.SemaphoreType.DMA,
        ],
    )
    out = k(x.reshape(-1), jnp.swapaxes(table, 0, 1))
    return out.reshape(BATCH, 1) + bias[None, :]
